# baseline (speedup 1.0000x reference)
import functools

import jax
import jax.numpy as jnp
from jax import lax
from jax.experimental import pallas as pl
from jax.experimental.pallas import tpu as pltpu
from jax.experimental.pallas import tpu_sc as plsc

N_NODES = 10000
N_EDGES = 320000
D = 128
L = 16
NC = 2
NS = 16
NW = NC * NS
E_PER_W = N_EDGES // NW
K = 40
NCHUNK = E_PER_W // K
NBUF = 5
ROWS_PER_TILE = 624
ROWS_TAIL = N_NODES - NS * ROWS_PER_TILE
ZCOPIES = ROWS_PER_TILE // K
ZREM = ROWS_PER_TILE - ZCOPIES * K

def _sc_gather_scatter(x, e4, w3):
    mesh = plsc.VectorSubcoreMesh(core_axis_name="c", subcore_axis_name="s")
    pshape = jax.ShapeDtypeStruct((N_NODES, D), jnp.float32)

    @functools.partial(
        pl.kernel,
        mesh=mesh,
        out_type=(pshape, pshape),
        scratch_types=[
            pltpu.VMEM((NBUF, 2, K), jnp.int32),
            pltpu.VMEM((NBUF, K), jnp.float32),
            pltpu.VMEM((NBUF, K, D), jnp.float32),
            pltpu.VMEM_SHARED((N_NODES, D), jnp.float32),
            pltpu.SemaphoreType.DMA((NBUF,)),
            pltpu.SemaphoreType.DMA((NBUF,)),
            pltpu.SemaphoreType.DMA((NBUF,)),
        ],
    )
    def k(x_hbm, e_hbm, w_hbm, p0_hbm, p1_hbm,
          idx_v, w_v, rows_v, h_sh, isem, gsem, ssem):
        c = lax.axis_index("c")
        s = lax.axis_index("s")
        wid = s * NC + c

        zero = jnp.zeros((L,), jnp.float32)

        def zrow(e, carry):
            for j in range(D // L):
                rows_v[0, e, pl.ds(j * L, L)] = zero
            return carry

        lax.fori_loop(0, K, zrow, 0)
        rbase = s * ROWS_PER_TILE
        for t in range(ZCOPIES):
            pltpu.sync_copy(rows_v.at[0], h_sh.at[pl.ds(rbase + t * K, K)])
        if ZREM:
            pltpu.sync_copy(rows_v.at[0, pl.ds(0, ZREM)],
                            h_sh.at[pl.ds(rbase + ZCOPIES * K, ZREM)])

        @pl.when(s == NS - 1)
        def _():
            pltpu.sync_copy(rows_v.at[0, pl.ds(0, ROWS_TAIL)],
                            h_sh.at[pl.ds(NS * ROWS_PER_TILE, ROWS_TAIL)])

        plsc.subcore_barrier()

        def s_idx_desc(ci, b):
            return pltpu.make_async_copy(
                e_hbm.at[0, wid, ci], idx_v.at[b, 0], isem.at[b])

        def d_idx_desc(ci, b):
            return pltpu.make_async_copy(
                e_hbm.at[1, wid, ci], idx_v.at[b, 1], isem.at[b])

        def w_desc(ci, b):
            return pltpu.make_async_copy(
                w_hbm.at[wid, ci], w_v.at[b], isem.at[b])

        def i_start(ci, b):
            s_idx_desc(ci, b).start()
            d_idx_desc(ci, b).start()
            w_desc(ci, b).start()

        def i_wait(ci, b):
            s_idx_desc(ci, b).wait()
            d_idx_desc(ci, b).wait()
            w_desc(ci, b).wait()

        def g_desc(ci, b):
            del ci
            return pltpu.make_async_copy(
                x_hbm.at[idx_v.at[b, 0]], rows_v.at[b], gsem.at[b])

        def s_desc(ci, b):
            del ci
            return pltpu.make_async_copy(
                rows_v.at[b], h_sh.at[idx_v.at[b, 1]], ssem.at[b])

        def edge_scale(b, e, w16, t):
            wb = jnp.full((L,), w16[t], jnp.float32)
            for j in range(D // L):
                sl = pl.ds(j * L, L)
                rows_v[b, e, sl] = rows_v[b, e, sl] * wb

        def scale(b):
            def egroup(g, carry):
                gbase = g * L
                w16 = w_v[b, pl.ds(gbase, L)]
                for t in range(L):
                    edge_scale(b, gbase + t, w16, t)
                return carry

            lax.fori_loop(0, K // L, egroup, 0)
            if K % L:
                w16 = w_v[b, pl.ds(K - L, L)]
                for t in range(L - (K % L), L):
                    edge_scale(b, (K - L) + t, w16, t)

        for u in range(3):
            i_start(u, u)
        for u in range(2):
            i_wait(u, u)
            g_desc(u, u).start()

        def chunk_step(ci, u):
            bp = (u + 3) % NBUF
            bg = (u + 2) % NBUF
            g_desc(ci, u).wait()
            scale(u)
            pltpu.async_copy(rows_v.at[u], h_sh.at[idx_v.at[u, 1]],
                             ssem.at[u], add=True)

            @pl.when((ci >= 2) & (ci + 3 < NCHUNK))
            def _():
                s_desc(ci - 2, bp).wait()

            @pl.when(ci + 3 < NCHUNK)
            def _():
                i_start(ci + 3, bp)

            @pl.when(ci + 2 < NCHUNK)
            def _():
                i_wait(ci + 2, bg)
                g_desc(ci + 2, bg).start()

        def body(t, carry):
            c0 = t * NBUF
            for u in range(NBUF):
                chunk_step(c0 + u, u)
            return carry

        lax.fori_loop(0, NCHUNK // NBUF, body, 0)

        for ci in range(NCHUNK - NBUF, NCHUNK):
            s_desc(ci, ci % NBUF).wait()

        plsc.subcore_barrier()

        @pl.when(c == 0)
        def _():
            pltpu.sync_copy(h_sh.at[pl.ds(rbase, ROWS_PER_TILE)],
                            p0_hbm.at[pl.ds(rbase, ROWS_PER_TILE)])

            @pl.when(s == NS - 1)
            def _():
                pltpu.sync_copy(h_sh.at[pl.ds(NS * ROWS_PER_TILE, ROWS_TAIL)],
                                p0_hbm.at[pl.ds(NS * ROWS_PER_TILE, ROWS_TAIL)])

        @pl.when(c == 1)
        def _():
            pltpu.sync_copy(h_sh.at[pl.ds(rbase, ROWS_PER_TILE)],
                            p1_hbm.at[pl.ds(rbase, ROWS_PER_TILE)])

            @pl.when(s == NS - 1)
            def _():
                pltpu.sync_copy(h_sh.at[pl.ds(NS * ROWS_PER_TILE, ROWS_TAIL)],
                                p1_hbm.at[pl.ds(NS * ROWS_PER_TILE, ROWS_TAIL)])

    return k(x, e4, w3)


R = 1000


def _linear_body(p0_ref, p1_ref, w_ref, b_ref, o_ref):
    h = p0_ref[...] + p1_ref[...]
    o_ref[...] = lax.dot_general(
        h, w_ref[...], (((1,), (1,)), ((), ())),
        preferred_element_type=jnp.float32) + b_ref[...]


def _tc_linear(p0, p1, W, b2d):
    return pl.pallas_call(
        _linear_body,
        grid=(N_NODES // R,),
        in_specs=[
            pl.BlockSpec((R, D), lambda i: (i, 0)),
            pl.BlockSpec((R, D), lambda i: (i, 0)),
            pl.BlockSpec((D, D), lambda i: (0, 0)),
            pl.BlockSpec((1, D), lambda i: (0, 0)),
        ],
        out_specs=pl.BlockSpec((R, D), lambda i: (i, 0)),
        out_shape=jax.ShapeDtypeStruct((N_NODES, D), jnp.float32),
    )(p0, p1, W, b2d)


def kernel(x, edge_index, edge_weight, W, b):
    e4 = edge_index.astype(jnp.int32).reshape(2, NW, NCHUNK, K)
    w3 = edge_weight.reshape(NW, NCHUNK, K)
    p0, p1 = _sc_gather_scatter(x, e4, w3)
    return _tc_linear(p0, p1, W, b.reshape(1, D))


# speedup vs baseline: 1.2954x; 1.2954x over previous
import functools

import jax
import jax.numpy as jnp
from jax import lax
from jax.experimental import pallas as pl
from jax.experimental.pallas import tpu as pltpu
from jax.experimental.pallas import tpu_sc as plsc

N_NODES = 10000
N_EDGES = 320000
D = 128
L = 16
NC = 2
NS = 16
NW = NC * NS
E_PER_W = N_EDGES // NW
K = 80
NCHUNK = E_PER_W // K
NBUF = 4
ROWS_PER_TILE = 624
ROWS_TAIL = N_NODES - NS * ROWS_PER_TILE
ZCOPIES = ROWS_PER_TILE // K
ZREM = ROWS_PER_TILE - ZCOPIES * K


def _sc_gather_scatter(x, e4, w3):
    mesh = plsc.VectorSubcoreMesh(core_axis_name="c", subcore_axis_name="s")
    pshape = jax.ShapeDtypeStruct((N_NODES, D), jnp.float32)

    @functools.partial(
        pl.kernel,
        mesh=mesh,
        out_type=(pshape, pshape),
        scratch_types=[
            pltpu.VMEM((NBUF, 2, K), jnp.int32),
            pltpu.VMEM((NBUF, K), jnp.float32),
            pltpu.VMEM((NBUF, K, D), jnp.float32),
            pltpu.VMEM_SHARED((N_NODES, D), jnp.float32),
            pltpu.SemaphoreType.DMA((NBUF,)),
            pltpu.SemaphoreType.DMA((NBUF,)),
            pltpu.SemaphoreType.DMA((NBUF,)),
        ],
    )
    def k(x_hbm, e_hbm, w_hbm, p0_hbm, p1_hbm,
          idx_v, w_v, rows_v, h_sh, isem, gsem, ssem):
        c = lax.axis_index("c")
        s = lax.axis_index("s")
        wid = s * NC + c

        zero = jnp.zeros((L,), jnp.float32)

        def zrow(e, carry):
            for j in range(D // L):
                rows_v[0, e, pl.ds(j * L, L)] = zero
            return carry

        lax.fori_loop(0, K, zrow, 0)
        rbase = s * ROWS_PER_TILE
        for t in range(ZCOPIES):
            pltpu.sync_copy(rows_v.at[0], h_sh.at[pl.ds(rbase + t * K, K)])
        if ZREM:
            pltpu.sync_copy(rows_v.at[0, pl.ds(0, ZREM)],
                            h_sh.at[pl.ds(rbase + ZCOPIES * K, ZREM)])

        @pl.when(s == NS - 1)
        def _():
            pltpu.sync_copy(rows_v.at[0, pl.ds(0, ROWS_TAIL)],
                            h_sh.at[pl.ds(NS * ROWS_PER_TILE, ROWS_TAIL)])

        plsc.subcore_barrier()

        def s_idx_desc(ci, b):
            return pltpu.make_async_copy(
                e_hbm.at[0, wid, ci], idx_v.at[b, 0], isem.at[b])

        def d_idx_desc(ci, b):
            return pltpu.make_async_copy(
                e_hbm.at[1, wid, ci], idx_v.at[b, 1], isem.at[b])

        def w_desc(ci, b):
            return pltpu.make_async_copy(
                w_hbm.at[wid, ci], w_v.at[b], isem.at[b])

        def i_start(ci, b):
            s_idx_desc(ci, b).start()
            d_idx_desc(ci, b).start()
            w_desc(ci, b).start()

        def i_wait(ci, b):
            s_idx_desc(ci, b).wait()
            d_idx_desc(ci, b).wait()
            w_desc(ci, b).wait()

        def g_desc(ci, b):
            del ci
            return pltpu.make_async_copy(
                x_hbm.at[idx_v.at[b, 0]], rows_v.at[b], gsem.at[b])

        def s_desc(ci, b):
            del ci
            return pltpu.make_async_copy(
                rows_v.at[b], h_sh.at[idx_v.at[b, 1]], ssem.at[b])

        def scale(b):
            def egroup(g, carry):
                w16 = w_v[b, pl.ds(g * L, L)]
                for t in range(L):
                    wb = jnp.full((L,), w16[t], jnp.float32)
                    e = g * L + t
                    for j in range(D // L):
                        sl = pl.ds(j * L, L)
                        rows_v[b, e, sl] = rows_v[b, e, sl] * wb
                return carry

            lax.fori_loop(0, K // L, egroup, 0)

        for u in range(3):
            i_start(u, u)
        for u in range(2):
            i_wait(u, u)
            g_desc(u, u).start()

        def chunk_step(ci, u):
            bp = (u + 3) % NBUF
            bg = (u + 2) % NBUF
            g_desc(ci, u).wait()
            scale(u)
            pltpu.async_copy(rows_v.at[u], h_sh.at[idx_v.at[u, 1]],
                             ssem.at[u], add=True)

            @pl.when(ci + 2 < NCHUNK)
            def _():
                i_wait(ci + 2, bg)
                g_desc(ci + 2, bg).start()

            @pl.when((ci >= 1) & (ci + 3 < NCHUNK))
            def _():
                s_desc(ci - 1, bp).wait()

            @pl.when(ci + 3 < NCHUNK)
            def _():
                i_start(ci + 3, bp)

        def body(t, carry):
            c0 = t * NBUF
            for u in range(NBUF):
                chunk_step(c0 + u, u)
            return carry

        lax.fori_loop(0, NCHUNK // NBUF, body, 0)
        chunk_step(NCHUNK - 1, (NCHUNK - 1) % NBUF)

        for ci in range(NCHUNK - NBUF, NCHUNK):
            s_desc(ci, ci % NBUF).wait()

        plsc.subcore_barrier()

        @pl.when(c == 0)
        def _():
            pltpu.sync_copy(h_sh.at[pl.ds(rbase, ROWS_PER_TILE)],
                            p0_hbm.at[pl.ds(rbase, ROWS_PER_TILE)])

            @pl.when(s == NS - 1)
            def _():
                pltpu.sync_copy(h_sh.at[pl.ds(NS * ROWS_PER_TILE, ROWS_TAIL)],
                                p0_hbm.at[pl.ds(NS * ROWS_PER_TILE, ROWS_TAIL)])

        @pl.when(c == 1)
        def _():
            pltpu.sync_copy(h_sh.at[pl.ds(rbase, ROWS_PER_TILE)],
                            p1_hbm.at[pl.ds(rbase, ROWS_PER_TILE)])

            @pl.when(s == NS - 1)
            def _():
                pltpu.sync_copy(h_sh.at[pl.ds(NS * ROWS_PER_TILE, ROWS_TAIL)],
                                p1_hbm.at[pl.ds(NS * ROWS_PER_TILE, ROWS_TAIL)])

    return k(x, e4, w3)


R = 2000


def _linear_body(p0_ref, p1_ref, w_ref, b_ref, o_ref):
    h = p0_ref[...] + p1_ref[...]
    o_ref[...] = lax.dot_general(
        h, w_ref[...], (((1,), (1,)), ((), ())),
        preferred_element_type=jnp.float32) + b_ref[...]


def _tc_linear(p0, p1, W, b2d):
    return pl.pallas_call(
        _linear_body,
        grid=(N_NODES // R,),
        in_specs=[
            pl.BlockSpec((R, D), lambda i: (i, 0)),
            pl.BlockSpec((R, D), lambda i: (i, 0)),
            pl.BlockSpec((D, D), lambda i: (0, 0)),
            pl.BlockSpec((1, D), lambda i: (0, 0)),
        ],
        out_specs=pl.BlockSpec((R, D), lambda i: (i, 0)),
        out_shape=jax.ShapeDtypeStruct((N_NODES, D), jnp.float32),
    )(p0, p1, W, b2d)


def kernel(x, edge_index, edge_weight, W, b):
    e4 = edge_index.astype(jnp.int32).reshape(2, NW, NCHUNK, K)
    w3 = edge_weight.reshape(NW, NCHUNK, K)
    p0, p1 = _sc_gather_scatter(x, e4, w3)
    return _tc_linear(p0, p1, W, b.reshape(1, D))


# speedup vs baseline: 1.3064x; 1.0085x over previous
import functools

import jax
import jax.numpy as jnp
from jax import lax
from jax.experimental import pallas as pl
from jax.experimental.pallas import tpu as pltpu
from jax.experimental.pallas import tpu_sc as plsc

N_NODES = 10000
N_EDGES = 320000
D = 128
L = 16
NC = 2
NS = 16
NW = NC * NS
E_PER_W = N_EDGES // NW
K = 80
NCHUNK = E_PER_W // K
NBUF = 4
ROWS_PER_TILE = 624
ROWS_TAIL = N_NODES - NS * ROWS_PER_TILE
ZCOPIES = ROWS_PER_TILE // K
ZREM = ROWS_PER_TILE - ZCOPIES * K


def _sc_gather_scatter(x, e4, w3):
    mesh = plsc.VectorSubcoreMesh(core_axis_name="c", subcore_axis_name="s")
    pshape = jax.ShapeDtypeStruct((N_NODES, D), jnp.float32)

    @functools.partial(
        pl.kernel,
        mesh=mesh,
        out_type=(pshape, pshape),
        scratch_types=[
            pltpu.VMEM((NBUF, 2, K), jnp.int32),
            pltpu.VMEM((NBUF, K), jnp.float32),
            pltpu.VMEM((NBUF, K, D), jnp.float32),
            pltpu.VMEM_SHARED((N_NODES, D), jnp.float32),
            pltpu.SemaphoreType.DMA((NBUF,)),
            pltpu.SemaphoreType.DMA((NBUF,)),
            pltpu.SemaphoreType.DMA((NBUF,)),
        ],
    )
    def k(x_hbm, e_hbm, w_hbm, p0_hbm, p1_hbm,
          idx_v, w_v, rows_v, h_sh, isem, gsem, ssem):
        c = lax.axis_index("c")
        s = lax.axis_index("s")
        wid = s * NC + c

        def s_idx_desc(ci, b):
            return pltpu.make_async_copy(
                e_hbm.at[0, wid, ci], idx_v.at[b, 0], isem.at[b])

        def d_idx_desc(ci, b):
            return pltpu.make_async_copy(
                e_hbm.at[1, wid, ci], idx_v.at[b, 1], isem.at[b])

        def w_desc(ci, b):
            return pltpu.make_async_copy(
                w_hbm.at[wid, ci], w_v.at[b], isem.at[b])

        def i_start(ci, b):
            s_idx_desc(ci, b).start()
            d_idx_desc(ci, b).start()
            w_desc(ci, b).start()

        def i_wait(ci, b):
            s_idx_desc(ci, b).wait()
            d_idx_desc(ci, b).wait()
            w_desc(ci, b).wait()

        def g_desc(ci, b):
            del ci
            return pltpu.make_async_copy(
                x_hbm.at[idx_v.at[b, 0]], rows_v.at[b], gsem.at[b])

        def s_desc(ci, b):
            del ci
            return pltpu.make_async_copy(
                rows_v.at[b], h_sh.at[idx_v.at[b, 1]], ssem.at[b])

        def scale(b):
            def egroup(g, carry):
                w16 = w_v[b, pl.ds(g * L, L)]
                for t in range(L):
                    wb = jnp.full((L,), w16[t], jnp.float32)
                    e = g * L + t
                    for j in range(D // L):
                        sl = pl.ds(j * L, L)
                        rows_v[b, e, sl] = rows_v[b, e, sl] * wb
                return carry

            lax.fori_loop(0, K // L, egroup, 0)

        for u in range(3):
            i_start(u, u)
        for u in range(2):
            i_wait(u, u)
            g_desc(u, u).start()

        zero = jnp.zeros((L,), jnp.float32)
        ZB = NBUF - 1

        def zrow(e, carry):
            for j in range(D // L):
                rows_v[ZB, e, pl.ds(j * L, L)] = zero
            return carry

        lax.fori_loop(0, K, zrow, 0)
        rbase = s * ROWS_PER_TILE
        for t in range(ZCOPIES):
            pltpu.sync_copy(rows_v.at[ZB], h_sh.at[pl.ds(rbase + t * K, K)])
        if ZREM:
            pltpu.sync_copy(rows_v.at[ZB, pl.ds(0, ZREM)],
                            h_sh.at[pl.ds(rbase + ZCOPIES * K, ZREM)])

        @pl.when(s == NS - 1)
        def _():
            pltpu.sync_copy(rows_v.at[ZB, pl.ds(0, ROWS_TAIL)],
                            h_sh.at[pl.ds(NS * ROWS_PER_TILE, ROWS_TAIL)])

        plsc.subcore_barrier()

        def chunk_step(ci, u):
            bp = (u + 3) % NBUF
            bg = (u + 2) % NBUF
            g_desc(ci, u).wait()
            scale(u)
            pltpu.async_copy(rows_v.at[u], h_sh.at[idx_v.at[u, 1]],
                             ssem.at[u], add=True)

            @pl.when(ci + 2 < NCHUNK)
            def _():
                i_wait(ci + 2, bg)
                g_desc(ci + 2, bg).start()

            @pl.when((ci >= 1) & (ci + 3 < NCHUNK))
            def _():
                s_desc(ci - 1, bp).wait()

            @pl.when(ci + 3 < NCHUNK)
            def _():
                i_start(ci + 3, bp)

        def body(t, carry):
            c0 = t * NBUF
            for u in range(NBUF):
                chunk_step(c0 + u, u)
            return carry

        lax.fori_loop(0, NCHUNK // NBUF, body, 0)
        chunk_step(NCHUNK - 1, (NCHUNK - 1) % NBUF)

        for ci in range(NCHUNK - NBUF, NCHUNK):
            s_desc(ci, ci % NBUF).wait()

        plsc.subcore_barrier()

        @pl.when(c == 0)
        def _():
            pltpu.sync_copy(h_sh.at[pl.ds(rbase, ROWS_PER_TILE)],
                            p0_hbm.at[pl.ds(rbase, ROWS_PER_TILE)])

            @pl.when(s == NS - 1)
            def _():
                pltpu.sync_copy(h_sh.at[pl.ds(NS * ROWS_PER_TILE, ROWS_TAIL)],
                                p0_hbm.at[pl.ds(NS * ROWS_PER_TILE, ROWS_TAIL)])

        @pl.when(c == 1)
        def _():
            pltpu.sync_copy(h_sh.at[pl.ds(rbase, ROWS_PER_TILE)],
                            p1_hbm.at[pl.ds(rbase, ROWS_PER_TILE)])

            @pl.when(s == NS - 1)
            def _():
                pltpu.sync_copy(h_sh.at[pl.ds(NS * ROWS_PER_TILE, ROWS_TAIL)],
                                p1_hbm.at[pl.ds(NS * ROWS_PER_TILE, ROWS_TAIL)])

    return k(x, e4, w3)


R = 2000


def _linear_body(p0_ref, p1_ref, w_ref, b_ref, o_ref):
    h = p0_ref[...] + p1_ref[...]
    o_ref[...] = lax.dot_general(
        h, w_ref[...], (((1,), (1,)), ((), ())),
        preferred_element_type=jnp.float32) + b_ref[...]


def _tc_linear(p0, p1, W, b2d):
    return pl.pallas_call(
        _linear_body,
        grid=(N_NODES // R,),
        in_specs=[
            pl.BlockSpec((R, D), lambda i: (i, 0)),
            pl.BlockSpec((R, D), lambda i: (i, 0)),
            pl.BlockSpec((D, D), lambda i: (0, 0)),
            pl.BlockSpec((1, D), lambda i: (0, 0)),
        ],
        out_specs=pl.BlockSpec((R, D), lambda i: (i, 0)),
        out_shape=jax.ShapeDtypeStruct((N_NODES, D), jnp.float32),
    )(p0, p1, W, b2d)


def kernel(x, edge_index, edge_weight, W, b):
    e4 = edge_index.astype(jnp.int32).reshape(2, NW, NCHUNK, K)
    w3 = edge_weight.reshape(NW, NCHUNK, K)
    p0, p1 = _sc_gather_scatter(x, e4, w3)
    return _tc_linear(p0, p1, W, b.reshape(1, D))


# speedup vs baseline: 1.3257x; 1.0148x over previous
import functools

import jax
import jax.numpy as jnp
from jax import lax
from jax.experimental import pallas as pl
from jax.experimental.pallas import tpu as pltpu
from jax.experimental.pallas import tpu_sc as plsc

N_NODES = 10000
N_EDGES = 320000
D = 128
L = 16
NC = 2
NS = 16
NW = NC * NS
E_PER_W = N_EDGES // NW
K = 80
NCHUNK = E_PER_W // K
NBUF = 4
ROWS_PER_TILE = 624
ROWS_TAIL = N_NODES - NS * ROWS_PER_TILE
ZCOPIES = ROWS_PER_TILE // K
ZREM = ROWS_PER_TILE - ZCOPIES * K


def _sc_gather_scatter(x, e4, w3):
    mesh = plsc.VectorSubcoreMesh(core_axis_name="c", subcore_axis_name="s")
    pshape = jax.ShapeDtypeStruct((N_NODES, D), jnp.float32)

    @functools.partial(
        pl.kernel,
        mesh=mesh,
        out_type=(pshape, pshape),
        scratch_types=[
            pltpu.VMEM((NBUF, 2, K), jnp.int32),
            pltpu.VMEM((NBUF, K), jnp.float32),
            pltpu.VMEM((NBUF, K, D), jnp.float32),
            pltpu.VMEM_SHARED((N_NODES, D), jnp.float32),
            pltpu.SemaphoreType.DMA((NBUF,)),
            pltpu.SemaphoreType.DMA((NBUF,)),
            pltpu.SemaphoreType.DMA((NBUF,)),
        ],
    )
    def k(x_hbm, e_hbm, w_hbm, p0_hbm, p1_hbm,
          idx_v, w_v, rows_v, h_sh, isem, gsem, ssem):
        c = lax.axis_index("c")
        s = lax.axis_index("s")
        wid = s * NC + c

        def s_idx_desc(ci, b):
            return pltpu.make_async_copy(
                e_hbm.at[0, wid, ci], idx_v.at[b, 0], isem.at[b])

        def d_idx_desc(ci, b):
            return pltpu.make_async_copy(
                e_hbm.at[1, wid, ci], idx_v.at[b, 1], isem.at[b])

        def w_desc(ci, b):
            return pltpu.make_async_copy(
                w_hbm.at[wid, ci], w_v.at[b], isem.at[b])

        def i_start(ci, b):
            s_idx_desc(ci, b).start()
            d_idx_desc(ci, b).start()
            w_desc(ci, b).start()

        def i_wait(ci, b):
            s_idx_desc(ci, b).wait()
            d_idx_desc(ci, b).wait()
            w_desc(ci, b).wait()

        def g_desc(ci, b):
            del ci
            return pltpu.make_async_copy(
                x_hbm.at[idx_v.at[b, 0]], rows_v.at[b], gsem.at[b])

        def s_desc(ci, b):
            del ci
            return pltpu.make_async_copy(
                rows_v.at[b], h_sh.at[idx_v.at[b, 1]], ssem.at[b])

        def scale(b):
            def egroup(g, carry):
                w16 = w_v[b, pl.ds(g * L, L)]
                for t in range(L):
                    wb = jnp.full((L,), w16[t], jnp.float32)
                    e = g * L + t
                    for j in range(D // L):
                        sl = pl.ds(j * L, L)
                        rows_v[b, e, sl] = rows_v[b, e, sl] * wb
                return carry

            lax.fori_loop(0, K // L, egroup, 0)

        for u in range(3):
            i_start(u, u)
        for u in range(2):
            i_wait(u, u)
            g_desc(u, u).start()

        zero = jnp.zeros((L,), jnp.float32)
        ZB = NBUF - 1

        def zrow(e, carry):
            for j in range(D // L):
                rows_v[ZB, e, pl.ds(j * L, L)] = zero
            return carry

        lax.fori_loop(0, K, zrow, 0)
        rbase = s * ROWS_PER_TILE
        for t in range(ZCOPIES):
            pltpu.sync_copy(rows_v.at[ZB], h_sh.at[pl.ds(rbase + t * K, K)])
        if ZREM:
            pltpu.sync_copy(rows_v.at[ZB, pl.ds(0, ZREM)],
                            h_sh.at[pl.ds(rbase + ZCOPIES * K, ZREM)])

        @pl.when(s == NS - 1)
        def _():
            pltpu.sync_copy(rows_v.at[ZB, pl.ds(0, ROWS_TAIL)],
                            h_sh.at[pl.ds(NS * ROWS_PER_TILE, ROWS_TAIL)])

        plsc.subcore_barrier()

        def chunk_step(ci, u):
            bp = (u + 3) % NBUF
            bg = (u + 2) % NBUF
            g_desc(ci, u).wait()
            scale(u)
            pltpu.async_copy(rows_v.at[u], h_sh.at[idx_v.at[u, 1]],
                             ssem.at[u], add=True)

            @pl.when(ci + 2 < NCHUNK)
            def _():
                i_wait(ci + 2, bg)
                g_desc(ci + 2, bg).start()

            @pl.when((ci >= 1) & (ci + 3 < NCHUNK))
            def _():
                s_desc(ci - 1, bp).wait()

            @pl.when(ci + 3 < NCHUNK)
            def _():
                i_start(ci + 3, bp)

        def body(t, carry):
            c0 = t * NBUF
            for u in range(NBUF):
                chunk_step(c0 + u, u)
            return carry

        lax.fori_loop(0, NCHUNK // NBUF, body, 0)
        chunk_step(NCHUNK - 1, (NCHUNK - 1) % NBUF)

        for ci in range(NCHUNK - NBUF, NCHUNK):
            s_desc(ci, ci % NBUF).wait()

        plsc.subcore_barrier()

        @pl.when(c == 0)
        def _():
            pltpu.sync_copy(h_sh.at[pl.ds(rbase, ROWS_PER_TILE)],
                            p0_hbm.at[pl.ds(rbase, ROWS_PER_TILE)])

            @pl.when(s == NS - 1)
            def _():
                pltpu.sync_copy(h_sh.at[pl.ds(NS * ROWS_PER_TILE, ROWS_TAIL)],
                                p0_hbm.at[pl.ds(NS * ROWS_PER_TILE, ROWS_TAIL)])

        @pl.when(c == 1)
        def _():
            pltpu.sync_copy(h_sh.at[pl.ds(rbase, ROWS_PER_TILE)],
                            p1_hbm.at[pl.ds(rbase, ROWS_PER_TILE)])

            @pl.when(s == NS - 1)
            def _():
                pltpu.sync_copy(h_sh.at[pl.ds(NS * ROWS_PER_TILE, ROWS_TAIL)],
                                p1_hbm.at[pl.ds(NS * ROWS_PER_TILE, ROWS_TAIL)])

    return k(x, e4, w3)


R = 5000


def _linear_body(p0_ref, p1_ref, w_ref, b_ref, o_ref):
    h = p0_ref[...] + p1_ref[...]
    o_ref[...] = lax.dot_general(
        h, w_ref[...], (((1,), (1,)), ((), ())),
        preferred_element_type=jnp.float32) + b_ref[...]


def _tc_linear(p0, p1, W, b2d):
    return pl.pallas_call(
        _linear_body,
        grid=(N_NODES // R,),
        in_specs=[
            pl.BlockSpec((R, D), lambda i: (i, 0)),
            pl.BlockSpec((R, D), lambda i: (i, 0)),
            pl.BlockSpec((D, D), lambda i: (0, 0)),
            pl.BlockSpec((1, D), lambda i: (0, 0)),
        ],
        out_specs=pl.BlockSpec((R, D), lambda i: (i, 0)),
        out_shape=jax.ShapeDtypeStruct((N_NODES, D), jnp.float32),
    )(p0, p1, W, b2d)


def kernel(x, edge_index, edge_weight, W, b):
    e4 = edge_index.astype(jnp.int32).reshape(2, NW, NCHUNK, K)
    w3 = edge_weight.reshape(NW, NCHUNK, K)
    p0, p1 = _sc_gather_scatter(x, e4, w3)
    return _tc_linear(p0, p1, W, b.reshape(1, D))
